# Initial kernel scaffold; baseline (speedup 1.0000x reference)
import jax
import jax.numpy as jnp
from jax.experimental import pallas as pl


def kernel(x, pos, edge_index, edge_attr, params):
    raise NotImplementedError("write your pallas kernel here")


# speedup vs baseline: 2.3829x; 2.3829x over previous
import functools

import jax
import jax.numpy as jnp
from jax import lax
from jax.experimental import pallas as pl
from jax.experimental.pallas import tpu as pltpu
from jax.experimental.pallas import tpu_sc as plsc

N = 10000
E = 320000
D = 128
A = 16
CDIM = 16

NW = 32
K = 128
EW = 10240
KB = EW // K
EP = NW * EW
NP = 10240
ROWS_PER_TILE = NP // 16

BE = 2048
BN = 1024

_MESH = plsc.VectorSubcoreMesh(core_axis_name="c", subcore_axis_name="s")


def _silu(x):
    return x * jax.nn.sigmoid(x)


def _gather_body(ha, hb, co, idxr3, idxc3, gha, ghb, gcr, gcc,
                 idx_r, idx_c, buf_a, buf_b, buf_cr, buf_cc, sem_g, sem_w):
    c = lax.axis_index("c")
    s = lax.axis_index("s")
    wid = s * 2 + c
    base = wid * EW
    pltpu.sync_copy(idxr3.at[wid], idx_r)
    pltpu.sync_copy(idxc3.at[wid], idx_c)

    def step(t, carry):
        g1 = pltpu.async_copy(ha.at[idx_r.at[t]], buf_a, sem_g)
        g2 = pltpu.async_copy(hb.at[idx_c.at[t]], buf_b, sem_g)
        g3 = pltpu.async_copy(co.at[idx_r.at[t]], buf_cr, sem_g)
        g4 = pltpu.async_copy(co.at[idx_c.at[t]], buf_cc, sem_g)
        g1.wait(); g2.wait(); g3.wait(); g4.wait()
        off = base + t * K
        w1 = pltpu.async_copy(buf_a, gha.at[pl.ds(off, K)], sem_w)
        w2 = pltpu.async_copy(buf_b, ghb.at[pl.ds(off, K)], sem_w)
        w3 = pltpu.async_copy(buf_cr, gcr.at[pl.ds(off, K)], sem_w)
        w4 = pltpu.async_copy(buf_cc, gcc.at[pl.ds(off, K)], sem_w)
        w1.wait(); w2.wait(); w3.wait(); w4.wait()
        return carry

    lax.fori_loop(0, KB, step, 0)


def _sc_gather(ha, hb, co, idxr3, idxc3):
    f32 = jnp.float32
    out_type = (
        jax.ShapeDtypeStruct((EP, D), f32),
        jax.ShapeDtypeStruct((EP, D), f32),
        jax.ShapeDtypeStruct((EP, CDIM), f32),
        jax.ShapeDtypeStruct((EP, CDIM), f32),
    )
    scratch = [
        pltpu.VMEM((KB, K), jnp.int32),
        pltpu.VMEM((KB, K), jnp.int32),
        pltpu.VMEM((K, D), f32),
        pltpu.VMEM((K, D), f32),
        pltpu.VMEM((K, CDIM), f32),
        pltpu.VMEM((K, CDIM), f32),
        pltpu.SemaphoreType.DMA,
        pltpu.SemaphoreType.DMA,
    ]
    fn = pl.kernel(_gather_body, out_type=out_type, mesh=_MESH,
                   scratch_types=scratch,
                   compiler_params=pltpu.CompilerParams(
                       use_tc_tiling_on_sc=False))
    return fn(ha, hb, co, idxr3, idxc3)


def _scatter_body(m, wcd, idxr3, hpart, cpart,
                  idx_r, m_buf, c_buf, acc_h, acc_c):
    c = lax.axis_index("c")
    s = lax.axis_index("s")
    wid = s * 2 + c
    base = wid * EW
    pltpu.sync_copy(idxr3.at[wid], idx_r)

    zero16 = jnp.zeros((16,), jnp.float32)

    def zrow(i, carry):
        for j in range(D // 16):
            m_buf[i, pl.ds(j * 16, 16)] = zero16
        c_buf[i, :] = zero16
        return carry

    lax.fori_loop(0, K, zrow, 0)
    tile_row0 = s * ROWS_PER_TILE
    for k in range(ROWS_PER_TILE // K):
        pltpu.sync_copy(m_buf, acc_h.at[pl.ds(tile_row0 + k * K, K)])
        pltpu.sync_copy(c_buf, acc_c.at[pl.ds(tile_row0 + k * K, K)])
    plsc.subcore_barrier()

    def step(t, carry):
        off = base + t * K
        pltpu.sync_copy(m.at[pl.ds(off, K)], m_buf)
        pltpu.sync_copy(wcd.at[pl.ds(off, K)], c_buf)
        pltpu.sync_copy(m_buf, acc_h.at[idx_r.at[t]], add=True)
        pltpu.sync_copy(c_buf, acc_c.at[idx_r.at[t]], add=True)
        return carry

    lax.fori_loop(0, KB, step, 0)
    plsc.subcore_barrier()
    pltpu.sync_copy(acc_h.at[pl.ds(tile_row0, ROWS_PER_TILE)],
                    hpart.at[c, pl.ds(tile_row0, ROWS_PER_TILE)])
    pltpu.sync_copy(acc_c.at[pl.ds(tile_row0, ROWS_PER_TILE)],
                    cpart.at[c, pl.ds(tile_row0, ROWS_PER_TILE)])


def _sc_scatter(m, wcd, idxr3):
    f32 = jnp.float32
    out_type = (
        jax.ShapeDtypeStruct((2, NP, D), f32),
        jax.ShapeDtypeStruct((2, NP, CDIM), f32),
    )
    scratch = [
        pltpu.VMEM((KB, K), jnp.int32),
        pltpu.VMEM((K, D), f32),
        pltpu.VMEM((K, CDIM), f32),
        pltpu.VMEM_SHARED((NP, D), f32),
        pltpu.VMEM_SHARED((NP, CDIM), f32),
    ]
    fn = pl.kernel(_scatter_body, out_type=out_type, mesh=_MESH,
                   scratch_types=scratch,
                   compiler_params=pltpu.CompilerParams(
                       use_tc_tiling_on_sc=False))
    return fn(m, wcd, idxr3)


def _full(shape):
    return pl.BlockSpec(shape, lambda i: tuple(0 for _ in shape))


def _edge_tc(gha, ghb, gcr, gcc, ea, We, W1, Wc0, bias):
    def body(gha_r, ghb_r, cr_r, cc_r, ea_r, We_r, W1_r, Wc0_r, b_r,
             m_r, wcd_r):
        cd = cr_r[...] - cc_r[...]
        radial = jnp.sum(cd * cd, axis=1, keepdims=True)
        b0 = b_r[0:1, :]
        b1 = b_r[1:2, :]
        bc0 = b_r[2:3, :]
        wc1 = b_r[3:4, :]
        wr = b_r[4:5, :]
        mpre = (gha_r[...] + ghb_r[...] + radial * wr + b0
                + jnp.dot(ea_r[...], We_r[...],
                          preferred_element_type=jnp.float32))
        m0 = _silu(mpre)
        m = _silu(jnp.dot(m0, W1_r[...], preferred_element_type=jnp.float32)
                  + b1)
        th = _silu(jnp.dot(m, Wc0_r[...], preferred_element_type=jnp.float32)
                   + bc0)
        t = jnp.sum(th * wc1, axis=1, keepdims=True)
        m_r[...] = m
        lane = lax.broadcasted_iota(jnp.int32, (BE, CDIM), 1)
        wcd_r[...] = jnp.where(lane == 3, 1.0, cd * t)

    grid = (EP // BE,)
    return pl.pallas_call(
        body,
        grid=grid,
        in_specs=[
            pl.BlockSpec((BE, D), lambda i: (i, 0)),
            pl.BlockSpec((BE, D), lambda i: (i, 0)),
            pl.BlockSpec((BE, CDIM), lambda i: (i, 0)),
            pl.BlockSpec((BE, CDIM), lambda i: (i, 0)),
            pl.BlockSpec((BE, A), lambda i: (i, 0)),
            _full((A, D)),
            _full((D, D)),
            _full((D, D)),
            _full((8, D)),
        ],
        out_specs=[
            pl.BlockSpec((BE, D), lambda i: (i, 0)),
            pl.BlockSpec((BE, CDIM), lambda i: (i, 0)),
        ],
        out_shape=[
            jax.ShapeDtypeStruct((EP, D), jnp.float32),
            jax.ShapeDtypeStruct((EP, CDIM), jnp.float32),
        ],
    )(gha, ghb, gcr, gcc, ea, We, W1, Wc0, bias)


def _node_tc(h, hp0, hp1, co, cp0, cp1, Wn0a, Wn0b, Wn1, Wax, Wbx, bias,
             last):
    def body(h_r, hp0_r, hp1_r, co_r, cp0_r, cp1_r,
             Wn0a_r, Wn0b_r, Wn1_r, Wax_r, Wbx_r, b_r, *outs):
        h = h_r[...]
        agg = hp0_r[...] + hp1_r[...]
        bn0 = b_r[0:1, :]
        bn1 = b_r[1:2, :]
        o = _silu(jnp.dot(h, Wn0a_r[...], preferred_element_type=jnp.float32)
                  + jnp.dot(agg, Wn0b_r[...],
                            preferred_element_type=jnp.float32) + bn0)
        o = jnp.dot(o, Wn1_r[...], preferred_element_type=jnp.float32) + bn1
        hn = h + o
        csum = cp0_r[...] + cp1_r[...]
        cnt = jnp.clip(csum[:, 3:4], 1.0, None)
        upd = csum / cnt
        lane = lax.broadcasted_iota(jnp.int32, (BN, CDIM), 1)
        co_new = co_r[...] + jnp.where(lane < 3, upd, 0.0)
        if last:
            hf_r, co_r_out = outs
            hf_r[...] = (jnp.dot(hn, Wax_r[...],
                                 preferred_element_type=jnp.float32)
                         + b_r[2:3, :])
            co_r_out[...] = co_new
        else:
            hn_r, co_r_out, ha_r, hb_r = outs
            hn_r[...] = hn
            co_r_out[...] = co_new
            ha_r[...] = jnp.dot(hn, Wax_r[...],
                                preferred_element_type=jnp.float32)
            hb_r[...] = jnp.dot(hn, Wbx_r[...],
                                preferred_element_type=jnp.float32)

    grid = (NP // BN,)
    nd = pl.BlockSpec((BN, D), lambda i: (i, 0))
    ndc = pl.BlockSpec((BN, CDIM), lambda i: (i, 0))
    if last:
        out_specs = [nd, ndc]
        out_shape = [jax.ShapeDtypeStruct((NP, D), jnp.float32),
                     jax.ShapeDtypeStruct((NP, CDIM), jnp.float32)]
    else:
        out_specs = [nd, ndc, nd, nd]
        out_shape = [jax.ShapeDtypeStruct((NP, D), jnp.float32),
                     jax.ShapeDtypeStruct((NP, CDIM), jnp.float32),
                     jax.ShapeDtypeStruct((NP, D), jnp.float32),
                     jax.ShapeDtypeStruct((NP, D), jnp.float32)]
    return pl.pallas_call(
        body,
        grid=grid,
        in_specs=[nd, nd, nd, ndc, ndc, ndc,
                  _full((D, D)), _full((D, D)), _full((D, D)),
                  _full((D, D)), _full((D, D)), _full((8, D))],
        out_specs=out_specs,
        out_shape=out_shape,
    )(h, hp0, hp1, co, cp0, cp1, Wn0a, Wn0b, Wn1, Wax, Wbx, bias)


def _init_tc(xp, Wemb, Wa0, Wb0, bias):
    def body(x_r, Wemb_r, Wa_r, Wb_r, b_r, h_r, ha_r, hb_r):
        h = (jnp.dot(x_r[...], Wemb_r[...],
                     preferred_element_type=jnp.float32) + b_r[0:1, :])
        h_r[...] = h
        ha_r[...] = jnp.dot(h, Wa_r[...], preferred_element_type=jnp.float32)
        hb_r[...] = jnp.dot(h, Wb_r[...], preferred_element_type=jnp.float32)

    grid = (NP // BN,)
    nd = pl.BlockSpec((BN, D), lambda i: (i, 0))
    return pl.pallas_call(
        body,
        grid=grid,
        in_specs=[nd, _full((D, D)), _full((D, D)), _full((D, D)),
                  _full((8, D))],
        out_specs=[nd, nd, nd],
        out_shape=[jax.ShapeDtypeStruct((NP, D), jnp.float32)] * 3,
    )(xp, Wemb, Wa0, Wb0, bias)


def _bias_stack(rows):
    stack = jnp.stack(rows, axis=0)
    return jnp.pad(stack, ((0, 8 - stack.shape[0]), (0, 0)))


def kernel(x, pos, edge_index, edge_attr, params):
    f32 = jnp.float32
    xp = jnp.pad(x.astype(f32), ((0, NP - N), (0, 0)))
    co = jnp.zeros((NP, CDIM), f32).at[:N, :3].set(pos.astype(f32))
    row = edge_index[0].astype(jnp.int32)
    col = edge_index[1].astype(jnp.int32)
    idxr3 = jnp.pad(row, (0, EP - E), constant_values=N).reshape(NW, KB, K)
    idxc3 = jnp.pad(col, (0, EP - E), constant_values=N).reshape(NW, KB, K)
    eap = jnp.pad(edge_attr.astype(f32), ((0, EP - E), (0, 0)))

    layers = params["layers"]
    l0 = layers[0]
    W0 = l0["edge_mlp0"]["W"]
    init_bias = _bias_stack([params["emb_in"]["b"]])
    h, ha, hb = _init_tc(xp, params["emb_in"]["W"],
                         W0[:D, :], W0[D:2 * D, :], init_bias)

    n_layers = len(layers)
    for li, lp in enumerate(layers):
        W0 = lp["edge_mlp0"]["W"]
        edge_bias = _bias_stack([
            lp["edge_mlp0"]["b"],
            lp["edge_mlp1"]["b"],
            lp["coord_mlp0"]["b"],
            lp["coord_mlp1"]["W"][:, 0],
            W0[2 * D, :],
        ])
        gha, ghb, gcr, gcc = _sc_gather(ha, hb, co, idxr3, idxc3)
        m, wcd = _edge_tc(gha, ghb, gcr, gcc, eap,
                          W0[2 * D + 1:, :], lp["edge_mlp1"]["W"],
                          lp["coord_mlp0"]["W"], edge_bias)
        hpart, cpart = _sc_scatter(m, wcd, idxr3)
        Wn = lp["node_mlp0"]["W"]
        last = li == n_layers - 1
        if last:
            node_bias = _bias_stack([lp["node_mlp0"]["b"],
                                     lp["node_mlp1"]["b"],
                                     params["emb_out"]["b"]])
            hf, co = _node_tc(h, hpart[0], hpart[1], co, cpart[0], cpart[1],
                              Wn[:D, :], Wn[D:, :], lp["node_mlp1"]["W"],
                              params["emb_out"]["W"], params["emb_out"]["W"],
                              node_bias, True)
        else:
            node_bias = _bias_stack([lp["node_mlp0"]["b"],
                                     lp["node_mlp1"]["b"]])
            Wnext = layers[li + 1]["edge_mlp0"]["W"]
            h, co, ha, hb = _node_tc(h, hpart[0], hpart[1], co,
                                     cpart[0], cpart[1],
                                     Wn[:D, :], Wn[D:, :],
                                     lp["node_mlp1"]["W"],
                                     Wnext[:D, :], Wnext[D:2 * D, :],
                                     node_bias, False)

    return co[:N, :3], hf[:N, :]


# speedup vs baseline: 3.0747x; 1.2903x over previous
import functools

import jax
import jax.numpy as jnp
from jax import lax
from jax.experimental import pallas as pl
from jax.experimental.pallas import tpu as pltpu
from jax.experimental.pallas import tpu_sc as plsc

N = 10000
E = 320000
D = 128
A = 16
CDIM = 16

NW = 32
K = 128
EW = 10240
KB = EW // K
EP = NW * EW
NP = 10240
ROWS_PER_TILE = NP // 16

BE = 2048
BN = 1024

_MESH = plsc.VectorSubcoreMesh(core_axis_name="c", subcore_axis_name="s")


def _silu(x):
    return x * jax.nn.sigmoid(x)


PHASES = 2
PKB = KB // PHASES


def _gather_body(ha, hb, co, idxr3, idxc3, gsum, gcd,
                 idx_r, idx_c,
                 g1_0, g2_0, g3_0, g4_0, o1_0, o2_0,
                 g1_1, g2_1, g3_1, g4_1, o1_1, o2_1,
                 sem_g0, sem_g1, sem_w0, sem_w1):
    c = lax.axis_index("c")
    s = lax.axis_index("s")
    wid = s * 2 + c
    base = wid * EW
    g1 = (g1_0, g1_1); g2 = (g2_0, g2_1)
    g3 = (g3_0, g3_1); g4 = (g4_0, g4_1)
    o1 = (o1_0, o1_1); o2 = (o2_0, o2_1)
    sem_g = (sem_g0, sem_g1)
    sem_w = (sem_w0, sem_w1)

    def fire_g(sl, t, b):
        pltpu.async_copy(ha.at[idx_r.at[sl]], g1[b], sem_g[b])
        pltpu.async_copy(hb.at[idx_c.at[sl]], g2[b], sem_g[b])
        pltpu.async_copy(co.at[idx_r.at[sl]], g3[b], sem_g[b])
        pltpu.async_copy(co.at[idx_c.at[sl]], g4[b], sem_g[b])

    def wait_g(sl, b):
        pltpu.make_async_copy(ha.at[idx_r.at[sl]], g1[b], sem_g[b]).wait()
        pltpu.make_async_copy(hb.at[idx_c.at[sl]], g2[b], sem_g[b]).wait()
        pltpu.make_async_copy(co.at[idx_r.at[sl]], g3[b], sem_g[b]).wait()
        pltpu.make_async_copy(co.at[idx_c.at[sl]], g4[b], sem_g[b]).wait()

    def fire_w(t, b):
        off = base + t * K
        pltpu.async_copy(o1[b], gsum.at[pl.ds(off, K)], sem_w[b])
        pltpu.async_copy(o2[b], gcd.at[pl.ds(off, K)], sem_w[b])

    def wait_w(t, b):
        off = base + t * K
        pltpu.make_async_copy(o1[b], gsum.at[pl.ds(off, K)],
                              sem_w[b]).wait()
        pltpu.make_async_copy(o2[b], gcd.at[pl.ds(off, K)],
                              sem_w[b]).wait()

    def compute(b):
        a, bb, cr, cc, oh, oc = g1[b], g2[b], g3[b], g4[b], o1[b], o2[b]

        def row(r, carry):
            for j in range(D // 16):
                sl = pl.ds(j * 16, 16)
                oh[r, sl] = a[r, sl] + bb[r, sl]
            oc[r, :] = cr[r, :] - cc[r, :]
            return carry

        lax.fori_loop(0, K, row, 0)

    for p in range(PHASES):
        base_t = p * PKB
        pltpu.sync_copy(idxr3.at[wid, pl.ds(base_t, PKB)], idx_r)
        pltpu.sync_copy(idxc3.at[wid, pl.ds(base_t, PKB)], idx_c)
        fire_g(0, base_t, 0)
        fire_g(1, base_t + 1, 1)

        def pair(i, do_wait_w, do_fire_g):
            for b in (0, 1):
                sl = 2 * i + b
                t = base_t + sl
                if do_wait_w:
                    wait_w(t - 2, b)
                wait_g(sl, b)
                compute(b)
                fire_w(t, b)
                if do_fire_g:
                    fire_g(sl + 2, t + 2, b)

        pair(0, False, True)

        def body(i, carry):
            pair(i, True, True)
            return carry

        lax.fori_loop(1, PKB // 2 - 1, body, 0)
        pair(PKB // 2 - 1, True, False)
        wait_w(base_t + PKB - 2, 0)
        wait_w(base_t + PKB - 1, 1)


def _sc_gather(ha, hb, co, idxr3, idxc3):
    f32 = jnp.float32
    out_type = (
        jax.ShapeDtypeStruct((EP, D), f32),
        jax.ShapeDtypeStruct((EP, CDIM), f32),
    )
    setbufs = [
        pltpu.VMEM((K, D), f32),
        pltpu.VMEM((K, D), f32),
        pltpu.VMEM((K, CDIM), f32),
        pltpu.VMEM((K, CDIM), f32),
        pltpu.VMEM((K, D), f32),
        pltpu.VMEM((K, CDIM), f32),
    ]
    scratch = ([pltpu.VMEM((PKB, K), jnp.int32),
                pltpu.VMEM((PKB, K), jnp.int32)]
               + setbufs + setbufs
               + [pltpu.SemaphoreType.DMA] * 4)
    fn = pl.kernel(_gather_body, out_type=out_type, mesh=_MESH,
                   scratch_types=scratch,
                   compiler_params=pltpu.CompilerParams(
                       use_tc_tiling_on_sc=False))
    return fn(ha, hb, co, idxr3, idxc3)


NSETS = 4


def _make_scatter_body(ncols, ksc, steps):
    def body(src, idx3, part, idx_r, b0, b1, b2, b3,
             acc, si0, si1, si2, si3, ss0, ss1, ss2, ss3):
        c = lax.axis_index("c")
        s = lax.axis_index("s")
        wid = s * 2 + c
        base = wid * EW
        pltpu.sync_copy(idx3.at[wid], idx_r)
        bufs = (b0, b1, b2, b3)
        sem_i = (si0, si1, si2, si3)
        sem_s = (ss0, ss1, ss2, ss3)

        zero16 = jnp.zeros((16,), jnp.float32)

        def zrow(i, carry):
            for j in range(ncols // 16):
                b0[i, pl.ds(j * 16, 16)] = zero16
            return carry

        lax.fori_loop(0, ksc, zrow, 0)
        tile_row0 = s * ROWS_PER_TILE
        for k in range(ROWS_PER_TILE // ksc):
            pltpu.sync_copy(b0, acc.at[pl.ds(tile_row0 + k * ksc, ksc)])
        plsc.subcore_barrier()

        def fire_in(t, b):
            pltpu.async_copy(src.at[pl.ds(base + t * ksc, ksc)],
                             bufs[b], sem_i[b])

        def wait_in(t, b):
            pltpu.make_async_copy(src.at[pl.ds(base + t * ksc, ksc)],
                                  bufs[b], sem_i[b]).wait()

        def fire_sc(sl, b):
            pltpu.async_copy(bufs[b], acc.at[idx_r.at[sl]], sem_s[b],
                             add=True)

        def wait_sc(sl, b):
            pltpu.make_async_copy(bufs[b], acc.at[idx_r.at[sl]],
                                  sem_s[b]).wait()

        fire_in(0, 0)

        def step(t, b, do_wait_sc, do_fire_in):
            bn = (b + 1) % NSETS
            if do_wait_sc:
                wait_sc(t + 1 - NSETS, bn)
            if do_fire_in:
                fire_in(t + 1, bn)
            wait_in(t, b)
            fire_sc(t, b)

        for t in range(NSETS):
            step(t, t, t == NSETS - 1, True)

        def loop(i, carry):
            for b in range(NSETS):
                step(NSETS * i + b, b, True, True)
            return carry

        lax.fori_loop(1, steps // NSETS - 1, loop, 0)
        for b in range(NSETS):
            t = steps - NSETS + b
            step(t, b, True, b < NSETS - 1)
        for b in range(1, NSETS):
            wait_sc(steps - NSETS + b, b)

        plsc.subcore_barrier()
        pltpu.sync_copy(acc.at[pl.ds(tile_row0, ROWS_PER_TILE)],
                        part.at[c, pl.ds(tile_row0, ROWS_PER_TILE)])

    return body


def _sc_scatter(src, idx3, ncols, ksc):
    f32 = jnp.float32
    steps = EW // ksc
    out_type = jax.ShapeDtypeStruct((2, NP, ncols), f32)
    scratch = ([pltpu.VMEM((steps, ksc), jnp.int32)]
               + [pltpu.VMEM((ksc, ncols), f32)] * NSETS
               + [pltpu.VMEM_SHARED((NP, ncols), f32)]
               + [pltpu.SemaphoreType.DMA] * (2 * NSETS))
    fn = pl.kernel(_make_scatter_body(ncols, ksc, steps), out_type=out_type,
                   mesh=_MESH, scratch_types=scratch,
                   compiler_params=pltpu.CompilerParams(
                       use_tc_tiling_on_sc=False))
    return fn(src, idx3)


def _full(shape):
    return pl.BlockSpec(shape, lambda i: tuple(0 for _ in shape))


def _edge_tc(gsum, gcd, ea, We, W1, Wc0, bias):
    def body(gsum_r, gcd_r, ea_r, We_r, W1_r, Wc0_r, b_r,
             m_r, wcd_r):
        cd = gcd_r[...]
        radial = jnp.sum(cd * cd, axis=1, keepdims=True)
        b0 = b_r[0:1, :]
        b1 = b_r[1:2, :]
        bc0 = b_r[2:3, :]
        wc1 = b_r[3:4, :]
        wr = b_r[4:5, :]
        mpre = (gsum_r[...] + radial * wr + b0
                + jnp.dot(ea_r[...], We_r[...],
                          preferred_element_type=jnp.float32))
        m0 = _silu(mpre)
        m = _silu(jnp.dot(m0, W1_r[...], preferred_element_type=jnp.float32)
                  + b1)
        th = _silu(jnp.dot(m, Wc0_r[...], preferred_element_type=jnp.float32)
                   + bc0)
        t = jnp.sum(th * wc1, axis=1, keepdims=True)
        m_r[...] = m
        lane = lax.broadcasted_iota(jnp.int32, (BE, CDIM), 1)
        wcd_r[...] = jnp.where(lane == 3, 1.0, cd * t)

    grid = (EP // BE,)
    return pl.pallas_call(
        body,
        grid=grid,
        in_specs=[
            pl.BlockSpec((BE, D), lambda i: (i, 0)),
            pl.BlockSpec((BE, CDIM), lambda i: (i, 0)),
            pl.BlockSpec((BE, A), lambda i: (i, 0)),
            _full((A, D)),
            _full((D, D)),
            _full((D, D)),
            _full((8, D)),
        ],
        out_specs=[
            pl.BlockSpec((BE, D), lambda i: (i, 0)),
            pl.BlockSpec((BE, CDIM), lambda i: (i, 0)),
        ],
        out_shape=[
            jax.ShapeDtypeStruct((EP, D), jnp.float32),
            jax.ShapeDtypeStruct((EP, CDIM), jnp.float32),
        ],
    )(gsum, gcd, ea, We, W1, Wc0, bias)


def _node_tc(h, hp0, hp1, co, cp0, cp1, Wn0a, Wn0b, Wn1, Wax, Wbx, bias,
             last):
    def body(h_r, hp0_r, hp1_r, co_r, cp0_r, cp1_r,
             Wn0a_r, Wn0b_r, Wn1_r, Wax_r, Wbx_r, b_r, *outs):
        h = h_r[...]
        agg = hp0_r[...] + hp1_r[...]
        bn0 = b_r[0:1, :]
        bn1 = b_r[1:2, :]
        o = _silu(jnp.dot(h, Wn0a_r[...], preferred_element_type=jnp.float32)
                  + jnp.dot(agg, Wn0b_r[...],
                            preferred_element_type=jnp.float32) + bn0)
        o = jnp.dot(o, Wn1_r[...], preferred_element_type=jnp.float32) + bn1
        hn = h + o
        csum = cp0_r[...] + cp1_r[...]
        cnt = jnp.clip(csum[:, 3:4], 1.0, None)
        upd = csum / cnt
        lane = lax.broadcasted_iota(jnp.int32, (BN, CDIM), 1)
        co_new = co_r[...] + jnp.where(lane < 3, upd, 0.0)
        if last:
            hf_r, co_r_out = outs
            hf_r[...] = (jnp.dot(hn, Wax_r[...],
                                 preferred_element_type=jnp.float32)
                         + b_r[2:3, :])
            co_r_out[...] = co_new
        else:
            hn_r, co_r_out, ha_r, hb_r = outs
            hn_r[...] = hn
            co_r_out[...] = co_new
            ha_r[...] = jnp.dot(hn, Wax_r[...],
                                preferred_element_type=jnp.float32)
            hb_r[...] = jnp.dot(hn, Wbx_r[...],
                                preferred_element_type=jnp.float32)

    grid = (NP // BN,)
    nd = pl.BlockSpec((BN, D), lambda i: (i, 0))
    ndc = pl.BlockSpec((BN, CDIM), lambda i: (i, 0))
    if last:
        out_specs = [nd, ndc]
        out_shape = [jax.ShapeDtypeStruct((NP, D), jnp.float32),
                     jax.ShapeDtypeStruct((NP, CDIM), jnp.float32)]
    else:
        out_specs = [nd, ndc, nd, nd]
        out_shape = [jax.ShapeDtypeStruct((NP, D), jnp.float32),
                     jax.ShapeDtypeStruct((NP, CDIM), jnp.float32),
                     jax.ShapeDtypeStruct((NP, D), jnp.float32),
                     jax.ShapeDtypeStruct((NP, D), jnp.float32)]
    return pl.pallas_call(
        body,
        grid=grid,
        in_specs=[nd, nd, nd, ndc, ndc, ndc,
                  _full((D, D)), _full((D, D)), _full((D, D)),
                  _full((D, D)), _full((D, D)), _full((8, D))],
        out_specs=out_specs,
        out_shape=out_shape,
    )(h, hp0, hp1, co, cp0, cp1, Wn0a, Wn0b, Wn1, Wax, Wbx, bias)


def _init_tc(xp, Wemb, Wa0, Wb0, bias):
    def body(x_r, Wemb_r, Wa_r, Wb_r, b_r, h_r, ha_r, hb_r):
        h = (jnp.dot(x_r[...], Wemb_r[...],
                     preferred_element_type=jnp.float32) + b_r[0:1, :])
        h_r[...] = h
        ha_r[...] = jnp.dot(h, Wa_r[...], preferred_element_type=jnp.float32)
        hb_r[...] = jnp.dot(h, Wb_r[...], preferred_element_type=jnp.float32)

    grid = (NP // BN,)
    nd = pl.BlockSpec((BN, D), lambda i: (i, 0))
    return pl.pallas_call(
        body,
        grid=grid,
        in_specs=[nd, _full((D, D)), _full((D, D)), _full((D, D)),
                  _full((8, D))],
        out_specs=[nd, nd, nd],
        out_shape=[jax.ShapeDtypeStruct((NP, D), jnp.float32)] * 3,
    )(xp, Wemb, Wa0, Wb0, bias)


def _bias_stack(rows):
    stack = jnp.stack(rows, axis=0)
    return jnp.pad(stack, ((0, 8 - stack.shape[0]), (0, 0)))


def kernel(x, pos, edge_index, edge_attr, params):
    f32 = jnp.float32
    xp = jnp.pad(x.astype(f32), ((0, NP - N), (0, 0)))
    co = jnp.zeros((NP, CDIM), f32).at[:N, :3].set(pos.astype(f32))
    row = edge_index[0].astype(jnp.int32)
    col = edge_index[1].astype(jnp.int32)
    rowp = jnp.pad(row, (0, EP - E), constant_values=N)
    idxr3 = rowp.reshape(NW, KB, K)
    idxr2 = rowp.reshape(NW, EW // 64, 64)
    idxc3 = jnp.pad(col, (0, EP - E), constant_values=N).reshape(NW, KB, K)
    eap = jnp.pad(edge_attr.astype(f32), ((0, EP - E), (0, 0)))

    layers = params["layers"]
    l0 = layers[0]
    W0 = l0["edge_mlp0"]["W"]
    init_bias = _bias_stack([params["emb_in"]["b"]])
    h, ha, hb = _init_tc(xp, params["emb_in"]["W"],
                         W0[:D, :], W0[D:2 * D, :], init_bias)

    n_layers = len(layers)
    for li, lp in enumerate(layers):
        W0 = lp["edge_mlp0"]["W"]
        edge_bias = _bias_stack([
            lp["edge_mlp0"]["b"],
            lp["edge_mlp1"]["b"],
            lp["coord_mlp0"]["b"],
            lp["coord_mlp1"]["W"][:, 0],
            W0[2 * D, :],
        ])
        gsum, gcd = _sc_gather(ha, hb, co, idxr3, idxc3)
        m, wcd = _edge_tc(gsum, gcd, eap,
                          W0[2 * D + 1:, :], lp["edge_mlp1"]["W"],
                          lp["coord_mlp0"]["W"], edge_bias)
        hpart = _sc_scatter(m, idxr2, D, 64)
        cpart = _sc_scatter(wcd, idxr3, CDIM, K)
        Wn = lp["node_mlp0"]["W"]
        last = li == n_layers - 1
        if last:
            node_bias = _bias_stack([lp["node_mlp0"]["b"],
                                     lp["node_mlp1"]["b"],
                                     params["emb_out"]["b"]])
            hf, co = _node_tc(h, hpart[0], hpart[1], co, cpart[0], cpart[1],
                              Wn[:D, :], Wn[D:, :], lp["node_mlp1"]["W"],
                              params["emb_out"]["W"], params["emb_out"]["W"],
                              node_bias, True)
        else:
            node_bias = _bias_stack([lp["node_mlp0"]["b"],
                                     lp["node_mlp1"]["b"]])
            Wnext = layers[li + 1]["edge_mlp0"]["W"]
            h, co, ha, hb = _node_tc(h, hpart[0], hpart[1], co,
                                     cpart[0], cpart[1],
                                     Wn[:D, :], Wn[D:, :],
                                     lp["node_mlp1"]["W"],
                                     Wnext[:D, :], Wnext[D:2 * D, :],
                                     node_bias, False)

    return co[:N, :3], hf[:N, :]
